# baseline (speedup 1.0000x reference)
import dataclasses
import functools

import jax
import jax.numpy as jnp
from jax import lax
from jax.experimental import pallas as pl
from jax.experimental.pallas import tpu as pltpu
from jax.experimental.pallas import tpu_sc as plsc

_VOCAB = 1000000
_EMBED = 16
_NUM_SAMPLED = 64
_BATCH = 16384

_NC = 2
_NSUB = 16
_NW = _NC * _NSUB

_PACK = 128 // _EMBED
_R_TILE = _BATCH // _NW
_BLK = 16
_RING = 4
_NBLK = _R_TILE // _BLK

_SAMP_PAD = _NW * _BLK
_BW = _BATCH + _SAMP_PAD
_BB_TILE = _BW // _NW

_CHUNK = 128


def _chunks(total):
  out = []
  c0 = 0
  while c0 < total:
    n = min(_CHUNK, total - c0)
    out.append((c0, n))
    c0 += n
  return out


def _sc_gather(emb, w, biases, base_e, sub_e, base_w, sub_w,
               base_s, sub_s, idx_b):
  mesh = plsc.VectorSubcoreMesh(core_axis_name="c", subcore_axis_name="s")
  cp = pltpu.CompilerParams()
  if "needs_layout_passes" in pltpu.CompilerParams.__dataclass_fields__:
    cp = dataclasses.replace(cp, needs_layout_passes=False)

  @functools.partial(
      pl.kernel,
      mesh=mesh,
      compiler_params=cp,
      out_type=(
          jax.ShapeDtypeStruct((_BATCH * _EMBED,), jnp.float32),
          jax.ShapeDtypeStruct((_BW * _EMBED,), jnp.float32),
          jax.ShapeDtypeStruct((_BW,), jnp.float32),
      ),
      scratch_types=[
          pltpu.VMEM((_R_TILE,), jnp.int32),
          pltpu.VMEM((_R_TILE,), jnp.int32),
          pltpu.VMEM((_R_TILE,), jnp.int32),
          pltpu.VMEM((_R_TILE,), jnp.int32),
          pltpu.VMEM((_BLK,), jnp.int32),
          pltpu.VMEM((_BLK,), jnp.int32),
          pltpu.VMEM((_BB_TILE,), jnp.int32),
          pltpu.VMEM((_RING * _BLK, _PACK, _EMBED), jnp.float32),
          pltpu.VMEM((_BLK, _PACK, _EMBED), jnp.float32),
          pltpu.VMEM((_R_TILE * _EMBED,), jnp.float32),
          pltpu.VMEM((_R_TILE * _EMBED,), jnp.float32),
          pltpu.VMEM((_BLK * _EMBED,), jnp.float32),
          pltpu.VMEM((_BB_TILE,), jnp.float32),
          pltpu.SemaphoreType.DMA,
          pltpu.SemaphoreType.DMA,
          pltpu.SemaphoreType.DMA,
          pltpu.SemaphoreType.DMA,
          pltpu.SemaphoreType.DMA,
          pltpu.SemaphoreType.DMA,
          pltpu.SemaphoreType.DMA,
      ],
  )
  def k(emb_hbm, w_hbm, b_hbm, basee_hbm, sube_hbm, basew_hbm, subw_hbm,
        bases_hbm, subs_hbm, idxb_hbm, out_e, out_w, out_b,
        basee_v, sube_v, basew_v, subw_v, bases_v, subs_v, idxb_v,
        ring, sbuf, ext_e, ext_w, ext_s, rows_b,
        sem0, sem1, sem2, sem3, semsmp, semb, semo):
    wid = lax.axis_index("s") * _NC + lax.axis_index("c")
    base_r = wid * _R_TILE
    base_bb = wid * _BB_TILE
    iota16 = lax.iota(jnp.int32, 16)
    sems = (sem0, sem1, sem2, sem3)

    pltpu.sync_copy(basee_hbm.at[pl.ds(base_r, _R_TILE)], basee_v)
    pltpu.sync_copy(sube_hbm.at[pl.ds(base_r, _R_TILE)], sube_v)
    pltpu.sync_copy(basew_hbm.at[pl.ds(base_r, _R_TILE)], basew_v)
    pltpu.sync_copy(subw_hbm.at[pl.ds(base_r, _R_TILE)], subw_v)
    pltpu.sync_copy(bases_hbm.at[pl.ds(wid * _BLK, _BLK)], bases_v)
    pltpu.sync_copy(subs_hbm.at[pl.ds(wid * _BLK, _BLK)], subs_v)
    pltpu.sync_copy(idxb_hbm.at[pl.ds(base_bb, _BB_TILE)], idxb_v)

    samp_bases = bases_v[...]

    @pl.loop(0, _BLK)
    def _(j):
      bj = pl.multiple_of(
          jnp.sum(jnp.where(iota16 == j, samp_bases, 0), axis=0), _PACK)
      pltpu.make_async_copy(
          w_hbm.at[pl.ds(bj, _PACK), :], sbuf.at[j], semsmp).start()

    bias_copies = []
    for c0, n in _chunks(_BB_TILE):
      bias_copies.append(pltpu.make_async_copy(
          b_hbm.at[idxb_v.at[pl.ds(c0, n)]], rows_b.at[pl.ds(c0, n)], semb))
    for cpy in bias_copies:
      cpy.start()

    def fire(table_hbm, base_v, bdyn, slot):
      bases16 = base_v[pl.ds(bdyn * _BLK, _BLK)]

      @pl.loop(0, _BLK)
      def _(j):
        bj = pl.multiple_of(
            jnp.sum(jnp.where(iota16 == j, bases16, 0), axis=0), _PACK)
        pltpu.make_async_copy(
            table_hbm.at[pl.ds(bj, _PACK), :],
            ring.at[slot * _BLK + j], sems[slot]).start()

    def drain(slot):
      pltpu.make_async_copy(
          emb_hbm.at[pl.ds(0, _BLK), :],
          ring.at[pl.ds(slot * _BLK, _BLK)], sems[slot]).wait()

    def extract(sub_v, src_buf, blk_base, bdyn, ext):
      sub16 = sub_v[pl.ds(bdyn * _BLK, _BLK)]
      blk_ids = iota16 + blk_base
      dst_base = iota16 * _EMBED + bdyn * (_BLK * _EMBED)

      @pl.loop(0, _EMBED)
      def _(kk):
        lane_k = jnp.full((16,), 0, jnp.int32) + kk
        vals = plsc.load_gather(src_buf, [blk_ids, sub16, lane_k])
        plsc.store_scatter(ext, [dst_base + kk], vals)

    def run_table(table_hbm, base_v, sub_v, ext):
      for r in range(_RING):
        fire(table_hbm, base_v, jnp.int32(r), r)

      @pl.loop(0, _NBLK // _RING)
      def _(phase):
        for r in range(_RING):
          b = phase * _RING + r
          drain(r)
          extract(sub_v, ring, r * _BLK, b, ext)

          @pl.when(phase < _NBLK // _RING - 1)
          def _():
            fire(table_hbm, base_v, b + _RING, r)

    run_table(emb_hbm, basee_v, sube_v, ext_e)
    w_e = pltpu.make_async_copy(
        ext_e, out_e.at[pl.ds(base_r * _EMBED, _R_TILE * _EMBED)], semo)
    w_e.start()

    run_table(w_hbm, basew_v, subw_v, ext_w)
    w_w = pltpu.make_async_copy(
        ext_w, out_w.at[pl.ds(base_r * _EMBED, _R_TILE * _EMBED)], semo)
    w_w.start()

    pltpu.make_async_copy(
        emb_hbm.at[pl.ds(0, _BLK), :], sbuf, semsmp).wait()
    sub16 = subs_v[...]
    dst_base = iota16 * _EMBED

    @pl.loop(0, _EMBED)
    def _(kk):
      lane_k = jnp.full((16,), 0, jnp.int32) + kk
      vals = plsc.load_gather(sbuf, [iota16, sub16, lane_k])
      plsc.store_scatter(ext_s, [dst_base + kk], vals)
    w_s = pltpu.make_async_copy(
        ext_s,
        out_w.at[pl.ds((_BATCH + wid * _BLK) * _EMBED, _BLK * _EMBED)], semo)
    w_s.start()

    for cpy in bias_copies:
      cpy.wait()
    w_b = pltpu.make_async_copy(
        rows_b, out_b.at[pl.ds(base_bb, _BB_TILE)], semo)
    w_b.start()
    w_e.wait()
    w_w.wait()
    w_s.wait()
    w_b.wait()

  return k(emb, w, biases, base_e, sub_e, base_w, sub_w, base_s, sub_s, idx_b)


def _tc_loss_body(embp_ref, wp_ref, bw_ref, bs_ref, labw_ref, samp_ref,
                  swt_ref, out_ref):
  embp = embp_ref[...]
  wp = wp_ref[...]
  bw = bw_ref[...]
  labs = labw_ref[...].astype(jnp.float32)
  samp = samp_ref[...].astype(jnp.float32)
  samp_b = bs_ref[...]
  swt = swt_ref[...]

  logv = jnp.log(float(_VOCAB) + 1.0)
  q_true = (jnp.log(labs + 2.0) - jnp.log(labs + 1.0)) / logv
  q_samp = (jnp.log(samp + 2.0) - jnp.log(samp + 1.0)) / logv

  i0 = lax.broadcasted_iota(jnp.int32, (128, 8), 0)
  i1 = lax.broadcasted_iota(jnp.int32, (128, 8), 1)
  m = jnp.where(i0 // _EMBED == i1, 1.0, 0.0)
  tl8 = lax.dot_general(embp * wp, m, (((1,), (0,)), ((), ())),
                        preferred_element_type=jnp.float32)
  tlw = tl8 + bw - jnp.log(_NUM_SAMPLED * q_true)

  zeros = jnp.zeros((_EMBED, _NUM_SAMPLED), jnp.float32)
  wbig = jnp.concatenate(
      [jnp.concatenate([swt if jj == j else zeros for jj in range(_PACK)],
                       axis=1) for j in range(_PACK)], axis=0)
  sl = lax.dot_general(embp, wbig, (((1,), (0,)), ((), ())),
                       preferred_element_type=jnp.float32)
  corr = samp_b - jnp.log(_NUM_SAMPLED * q_samp)
  sl = sl + jnp.concatenate([corr] * _PACK, axis=1)

  total = jnp.sum(jax.nn.softplus(-tlw)) + jnp.sum(jax.nn.softplus(sl))
  out_ref[...] = jnp.reshape(total / float(_BATCH), (1, 1))


def kernel(inputs, train_labels, embeddings, nce_weights, nce_biases):
  idx_e = inputs.astype(jnp.int32)
  labels = train_labels[:, 0].astype(jnp.int32)
  skey = jax.random.key(12345)
  sampled = jax.random.randint(skey, (_NUM_SAMPLED,), 0, _VOCAB).astype(
      jnp.int32)
  samp_padded = jnp.zeros((_SAMP_PAD,), jnp.int32).at[0:_NUM_SAMPLED].set(
      sampled)
  idx_b = jnp.concatenate([labels, samp_padded])

  base_e = (idx_e // _PACK) * _PACK
  sub_e = idx_e % _PACK
  base_w = (labels // _PACK) * _PACK
  sub_w = labels % _PACK
  base_s = (samp_padded // _PACK) * _PACK
  sub_s = samp_padded % _PACK

  embed_flat, w_flat, b_vals = _sc_gather(
      embeddings, nce_weights, nce_biases,
      base_e, sub_e, base_w, sub_w, base_s, sub_s, idx_b)
  embed = embed_flat.reshape(_BATCH, _EMBED)

  embp = embed_flat.reshape(_BATCH * _EMBED // 128, 128)
  wp_true = w_flat[: _BATCH * _EMBED].reshape(_BATCH * _EMBED // 128, 128)
  bw = b_vals[:_BATCH].reshape(_BATCH // _PACK, _PACK)
  bs = b_vals[_BATCH:_BATCH + _NUM_SAMPLED].reshape(1, _NUM_SAMPLED)
  labw = labels.reshape(_BATCH // _PACK, _PACK)
  swt = w_flat[_BATCH * _EMBED:(_BATCH + _NUM_SAMPLED) * _EMBED].reshape(
      _NUM_SAMPLED, _EMBED).T

  nce_cost = pl.pallas_call(
      _tc_loss_body,
      out_shape=jax.ShapeDtypeStruct((1, 1), jnp.float32),
  )(embp, wp_true, bw, bs, labw, sampled.reshape(1, _NUM_SAMPLED), swt)[0, 0]

  return embed, nce_cost


# speedup vs baseline: 3.4317x; 3.4317x over previous
import dataclasses
import functools

import jax
import jax.numpy as jnp
from jax import lax
from jax.experimental import pallas as pl
from jax.experimental.pallas import tpu as pltpu
from jax.experimental.pallas import tpu_sc as plsc

_VOCAB = 1000000
_EMBED = 16
_NUM_SAMPLED = 64
_BATCH = 16384

_NC = 2
_NSUB = 16
_NW = _NC * _NSUB

_PACK = 128 // _EMBED
_LANE = 128
_R_TILE = _BATCH // _NW
_BLK = 16
_RING = 2
_NBLK = _R_TILE // _BLK

_SAMP_PAD = _NW * _BLK
_BW = _BATCH + _SAMP_PAD
_BB_TILE = _BW // _NW

_CHUNK = 128


def _chunks(total):
  out = []
  c0 = 0
  while c0 < total:
    n = min(_CHUNK, total - c0)
    out.append((c0, n))
    c0 += n
  return out


def _sc_gather(emb, w, biases, base_e, sub_e, base_w, sub_w,
               base_s, sub_s, idx_b):
  mesh = plsc.VectorSubcoreMesh(core_axis_name="c", subcore_axis_name="s")
  cp = pltpu.CompilerParams()
  if "needs_layout_passes" in pltpu.CompilerParams.__dataclass_fields__:
    cp = dataclasses.replace(cp, needs_layout_passes=False)

  @functools.partial(
      pl.kernel,
      mesh=mesh,
      compiler_params=cp,
      out_type=(
          jax.ShapeDtypeStruct((_BATCH * _EMBED,), jnp.float32),
          jax.ShapeDtypeStruct((_BW * _EMBED,), jnp.float32),
          jax.ShapeDtypeStruct((_BW,), jnp.float32),
      ),
      scratch_types=[
          pltpu.VMEM((_R_TILE,), jnp.int32),
          pltpu.VMEM((_R_TILE,), jnp.int32),
          pltpu.VMEM((_R_TILE,), jnp.int32),
          pltpu.VMEM((_R_TILE,), jnp.int32),
          pltpu.VMEM((_BLK,), jnp.int32),
          pltpu.VMEM((_BLK,), jnp.int32),
          pltpu.VMEM((_BB_TILE,), jnp.int32),
          pltpu.VMEM((_RING * _BLK, _EMBED, _LANE), jnp.float32),
          pltpu.VMEM((_R_TILE * _EMBED,), jnp.float32),
          pltpu.VMEM((_R_TILE * _EMBED,), jnp.float32),
          pltpu.VMEM((_BLK * _EMBED,), jnp.float32),
          pltpu.VMEM((_BB_TILE,), jnp.float32),
          pltpu.SemaphoreType.DMA,
          pltpu.SemaphoreType.DMA,
          pltpu.SemaphoreType.DMA,
          pltpu.SemaphoreType.DMA,
          pltpu.SemaphoreType.DMA,
          pltpu.SemaphoreType.DMA,
          pltpu.SemaphoreType.DMA,
      ],
  )
  def k(emb_hbm, w_hbm, b_hbm, basee_hbm, sube_hbm, basew_hbm, subw_hbm,
        bases_hbm, subs_hbm, idxb_hbm, out_e, out_w, out_b,
        basee_v, sube_v, basew_v, subw_v, bases_v, subs_v, idxb_v,
        ring, ext_e, ext_w, ext_s, rows_b,
        sem0, sem1, sem2, sem3, semsmp, semb, semo):
    wid = lax.axis_index("s") * _NC + lax.axis_index("c")
    base_r = wid * _R_TILE
    base_bb = wid * _BB_TILE
    iota16 = lax.iota(jnp.int32, 16)
    sems = (sem0, sem1, sem2, sem3)

    pltpu.sync_copy(basee_hbm.at[pl.ds(base_r, _R_TILE)], basee_v)
    pltpu.sync_copy(sube_hbm.at[pl.ds(base_r, _R_TILE)], sube_v)
    pltpu.sync_copy(basew_hbm.at[pl.ds(base_r, _R_TILE)], basew_v)
    pltpu.sync_copy(subw_hbm.at[pl.ds(base_r, _R_TILE)], subw_v)
    pltpu.sync_copy(bases_hbm.at[pl.ds(wid * _BLK, _BLK)], bases_v)
    pltpu.sync_copy(subs_hbm.at[pl.ds(wid * _BLK, _BLK)], subs_v)
    pltpu.sync_copy(idxb_hbm.at[pl.ds(base_bb, _BB_TILE)], idxb_v)

    bias_copies = []
    for c0, n in _chunks(_BB_TILE):
      bias_copies.append(pltpu.make_async_copy(
          b_hbm.at[idxb_v.at[pl.ds(c0, n)]], rows_b.at[pl.ds(c0, n)], semb))
    for cpy in bias_copies:
      cpy.start()

    def fire(table_hbm, base_v, bdyn, slot):
      bases16 = base_v[pl.ds(bdyn * _BLK, _BLK)]

      @pl.loop(0, _BLK)
      def _(j):
        bj = pl.multiple_of(
            jnp.sum(jnp.where(iota16 == j, bases16, 0), axis=0), _LANE)
        pltpu.make_async_copy(
            table_hbm.at[:, pl.ds(bj, _LANE)],
            ring.at[slot * _BLK + j], sems[slot]).start()

    def drain(slot):
      pltpu.make_async_copy(
          emb_hbm.at[:, pl.ds(0, _LANE * _BLK)],
          ring.at[pl.ds(slot * _BLK, _BLK)], sems[slot]).wait()

    def extract(sub_v, src_buf, blk_base, bdyn, ext):
      sub16 = sub_v[pl.ds(bdyn * _BLK, _BLK)]
      blk_ids = iota16 + blk_base
      dst_base = iota16 * _EMBED + bdyn * (_BLK * _EMBED)

      @pl.loop(0, _EMBED)
      def _(kk):
        lane_k = jnp.full((16,), 0, jnp.int32) + kk
        vals = plsc.load_gather(src_buf, [blk_ids, lane_k, sub16])
        plsc.store_scatter(ext, [dst_base + kk], vals)

    def run_table(table_hbm, base_v, sub_v, ext):
      for r in range(_RING):
        fire(table_hbm, base_v, jnp.int32(r), r)

      @pl.loop(0, _NBLK // _RING)
      def _(phase):
        for r in range(_RING):
          b = phase * _RING + r
          drain(r)
          extract(sub_v, ring, r * _BLK, b, ext)

          @pl.when(phase < _NBLK // _RING - 1)
          def _():
            fire(table_hbm, base_v, b + _RING, r)

    run_table(emb_hbm, basee_v, sube_v, ext_e)
    w_e = pltpu.make_async_copy(
        ext_e, out_e.at[pl.ds(base_r * _EMBED, _R_TILE * _EMBED)], semo)
    w_e.start()

    run_table(w_hbm, basew_v, subw_v, ext_w)
    w_w = pltpu.make_async_copy(
        ext_w, out_w.at[pl.ds(base_r * _EMBED, _R_TILE * _EMBED)], semo)
    w_w.start()

    samp_bases = bases_v[...]

    @pl.loop(0, _BLK)
    def _(j):
      bj = pl.multiple_of(
          jnp.sum(jnp.where(iota16 == j, samp_bases, 0), axis=0), _LANE)
      pltpu.make_async_copy(
          w_hbm.at[:, pl.ds(bj, _LANE)], ring.at[j], semsmp).start()
    pltpu.make_async_copy(
        emb_hbm.at[:, pl.ds(0, _LANE * _BLK)],
        ring.at[pl.ds(0, _BLK)], semsmp).wait()
    sub16 = subs_v[...]
    dst_base = iota16 * _EMBED

    @pl.loop(0, _EMBED)
    def _(kk):
      lane_k = jnp.full((16,), 0, jnp.int32) + kk
      vals = plsc.load_gather(ring, [iota16, lane_k, sub16])
      plsc.store_scatter(ext_s, [dst_base + kk], vals)
    w_s = pltpu.make_async_copy(
        ext_s,
        out_w.at[pl.ds((_BATCH + wid * _BLK) * _EMBED, _BLK * _EMBED)], semo)
    w_s.start()

    for cpy in bias_copies:
      cpy.wait()
    w_b = pltpu.make_async_copy(
        rows_b, out_b.at[pl.ds(base_bb, _BB_TILE)], semo)
    w_b.start()
    w_e.wait()
    w_w.wait()
    w_s.wait()
    w_b.wait()

  return k(emb, w, biases, base_e, sub_e, base_w, sub_w, base_s, sub_s, idx_b)


def _tc_loss_body(embp_ref, wp_ref, bw_ref, bs_ref, labw_ref, samp_ref,
                  swt_ref, out_ref):
  embp = embp_ref[...]
  wp = wp_ref[...]
  bw = bw_ref[...]
  labs = labw_ref[...].astype(jnp.float32)
  samp = samp_ref[...].astype(jnp.float32)
  samp_b = bs_ref[...]
  swt = swt_ref[...]

  logv = jnp.log(float(_VOCAB) + 1.0)
  q_true = (jnp.log(labs + 2.0) - jnp.log(labs + 1.0)) / logv
  q_samp = (jnp.log(samp + 2.0) - jnp.log(samp + 1.0)) / logv

  i0 = lax.broadcasted_iota(jnp.int32, (128, 8), 0)
  i1 = lax.broadcasted_iota(jnp.int32, (128, 8), 1)
  m = jnp.where(i0 // _EMBED == i1, 1.0, 0.0)
  tl8 = lax.dot_general(embp * wp, m, (((1,), (0,)), ((), ())),
                        preferred_element_type=jnp.float32)
  tlw = tl8 + bw - jnp.log(_NUM_SAMPLED * q_true)

  zeros = jnp.zeros((_EMBED, _NUM_SAMPLED), jnp.float32)
  wbig = jnp.concatenate(
      [jnp.concatenate([swt if jj == j else zeros for jj in range(_PACK)],
                       axis=1) for j in range(_PACK)], axis=0)
  sl = lax.dot_general(embp, wbig, (((1,), (0,)), ((), ())),
                       preferred_element_type=jnp.float32)
  corr = samp_b - jnp.log(_NUM_SAMPLED * q_samp)
  sl = sl + jnp.concatenate([corr] * _PACK, axis=1)

  total = jnp.sum(jax.nn.softplus(-tlw)) + jnp.sum(jax.nn.softplus(sl))
  out_ref[...] = jnp.reshape(total / float(_BATCH), (1, 1))


def kernel(inputs, train_labels, embeddings, nce_weights, nce_biases):
  idx_e = inputs.astype(jnp.int32)
  labels = train_labels[:, 0].astype(jnp.int32)
  skey = jax.random.key(12345)
  sampled = jax.random.randint(skey, (_NUM_SAMPLED,), 0, _VOCAB).astype(
      jnp.int32)
  samp_padded = jnp.zeros((_SAMP_PAD,), jnp.int32).at[0:_NUM_SAMPLED].set(
      sampled)
  idx_b = jnp.concatenate([labels, samp_padded])

  base_e = (idx_e // _LANE) * _LANE
  sub_e = idx_e % _LANE
  base_w = (labels // _LANE) * _LANE
  sub_w = labels % _LANE
  base_s = (samp_padded // _LANE) * _LANE
  sub_s = samp_padded % _LANE

  embed_flat, w_flat, b_vals = _sc_gather(
      embeddings.T, nce_weights.T, nce_biases,
      base_e, sub_e, base_w, sub_w, base_s, sub_s, idx_b)
  embed = embed_flat.reshape(_BATCH, _EMBED)

  embp = embed_flat.reshape(_BATCH * _EMBED // 128, 128)
  wp_true = w_flat[: _BATCH * _EMBED].reshape(_BATCH * _EMBED // 128, 128)
  bw = b_vals[:_BATCH].reshape(_BATCH // _PACK, _PACK)
  bs = b_vals[_BATCH:_BATCH + _NUM_SAMPLED].reshape(1, _NUM_SAMPLED)
  labw = labels.reshape(_BATCH // _PACK, _PACK)
  swt = w_flat[_BATCH * _EMBED:(_BATCH + _NUM_SAMPLED) * _EMBED].reshape(
      _NUM_SAMPLED, _EMBED).T

  nce_cost = pl.pallas_call(
      _tc_loss_body,
      out_shape=jax.ShapeDtypeStruct((1, 1), jnp.float32),
  )(embp, wp_true, bw, bs, labw, sampled.reshape(1, _NUM_SAMPLED), swt)[0, 0]

  return embed, nce_cost


# speedup vs baseline: 3.8739x; 1.1289x over previous
import dataclasses
import functools

import jax
import jax.numpy as jnp
from jax import lax
from jax.experimental import pallas as pl
from jax.experimental.pallas import tpu as pltpu
from jax.experimental.pallas import tpu_sc as plsc

_VOCAB = 1000000
_EMBED = 16
_NUM_SAMPLED = 64
_BATCH = 16384

_NC = 2
_NSUB = 16
_NW = _NC * _NSUB

_PACK = 128 // _EMBED
_LANE = 128
_R_TILE = _BATCH // _NW
_BLK = 8
_RING = 2
_NBLK = _R_TILE // _BLK

_SAMP_PAD = _NW * _BLK
_BW = _BATCH + _SAMP_PAD
_BB_TILE = _BW // _NW

_CHUNK = 128


def _chunks(total):
  out = []
  c0 = 0
  while c0 < total:
    n = min(_CHUNK, total - c0)
    out.append((c0, n))
    c0 += n
  return out


def _sc_gather(emb, w, biases, base_e, sub_e, base_w, sub_w,
               base_s, sub_s, idx_b):
  mesh = plsc.VectorSubcoreMesh(core_axis_name="c", subcore_axis_name="s")
  cp = pltpu.CompilerParams()
  if "needs_layout_passes" in pltpu.CompilerParams.__dataclass_fields__:
    cp = dataclasses.replace(cp, needs_layout_passes=False)

  @functools.partial(
      pl.kernel,
      mesh=mesh,
      compiler_params=cp,
      out_type=(
          jax.ShapeDtypeStruct((_BATCH * _EMBED,), jnp.float32),
          jax.ShapeDtypeStruct((_BW * _EMBED,), jnp.float32),
          jax.ShapeDtypeStruct((_BW,), jnp.float32),
      ),
      scratch_types=[
          pltpu.VMEM((_R_TILE + 16,), jnp.int32),
          pltpu.VMEM((_R_TILE + 16,), jnp.int32),
          pltpu.VMEM((_R_TILE + 16,), jnp.int32),
          pltpu.VMEM((_R_TILE + 16,), jnp.int32),
          pltpu.VMEM((16,), jnp.int32),
          pltpu.VMEM((16,), jnp.int32),
          pltpu.VMEM((_BB_TILE,), jnp.int32),
          pltpu.VMEM((_RING * _BLK, _EMBED, _LANE), jnp.float32),
          pltpu.VMEM((_RING * _BLK, _EMBED, _LANE), jnp.float32),
          pltpu.VMEM((_R_TILE * _EMBED,), jnp.float32),
          pltpu.VMEM((_R_TILE * _EMBED,), jnp.float32),
          pltpu.VMEM((_BLK * _EMBED,), jnp.float32),
          pltpu.VMEM((_BB_TILE,), jnp.float32),
          pltpu.SemaphoreType.DMA,
          pltpu.SemaphoreType.DMA,
          pltpu.SemaphoreType.DMA,
          pltpu.SemaphoreType.DMA,
          pltpu.SemaphoreType.DMA,
          pltpu.SemaphoreType.DMA,
          pltpu.SemaphoreType.DMA,
      ],
  )
  def k(emb_hbm, w_hbm, b_hbm, basee_hbm, sube_hbm, basew_hbm, subw_hbm,
        bases_hbm, subs_hbm, idxb_hbm, out_e, out_w, out_b,
        basee_v, sube_v, basew_v, subw_v, bases_v, subs_v, idxb_v,
        ring_e, ring_w, ext_e, ext_w, ext_s, rows_b,
        sem0, sem1, sem2, sem3, semsmp, semb, semo):
    wid = lax.axis_index("s") * _NC + lax.axis_index("c")
    base_r = wid * _R_TILE
    base_bb = wid * _BB_TILE
    iota16 = lax.iota(jnp.int32, 16)
    sems = (sem0, sem1, sem2, sem3)

    pltpu.sync_copy(basee_hbm.at[pl.ds(base_r, _R_TILE)],
                    basee_v.at[pl.ds(0, _R_TILE)])
    pltpu.sync_copy(sube_hbm.at[pl.ds(base_r, _R_TILE)],
                    sube_v.at[pl.ds(0, _R_TILE)])
    pltpu.sync_copy(basew_hbm.at[pl.ds(base_r, _R_TILE)],
                    basew_v.at[pl.ds(0, _R_TILE)])
    pltpu.sync_copy(subw_hbm.at[pl.ds(base_r, _R_TILE)],
                    subw_v.at[pl.ds(0, _R_TILE)])
    pltpu.sync_copy(bases_hbm.at[pl.ds(wid * _BLK, _BLK)],
                    bases_v.at[pl.ds(0, _BLK)])
    pltpu.sync_copy(subs_hbm.at[pl.ds(wid * _BLK, _BLK)],
                    subs_v.at[pl.ds(0, _BLK)])
    pltpu.sync_copy(idxb_hbm.at[pl.ds(base_bb, _BB_TILE)], idxb_v)

    bias_copies = []
    for c0, n in _chunks(_BB_TILE):
      bias_copies.append(pltpu.make_async_copy(
          b_hbm.at[idxb_v.at[pl.ds(c0, n)]], rows_b.at[pl.ds(c0, n)], semb))
    for cpy in bias_copies:
      cpy.start()

    m8 = iota16 < _BLK

    def fire(table_hbm, base_v, ringbuf, sem_pair, bdyn, slot):
      bases16 = base_v[pl.ds(bdyn * _BLK, 16)]
      for j in range(_BLK):
        bj = pl.multiple_of(
            jnp.sum(jnp.where(iota16 == j, bases16, 0), axis=0), _LANE)
        pltpu.make_async_copy(
            table_hbm.at[:, pl.ds(bj, _LANE)],
            ringbuf.at[slot * _BLK + j], sem_pair[slot]).start()

    def drain(ringbuf, sem_pair, slot):
      pltpu.make_async_copy(
          emb_hbm.at[:, pl.ds(0, _LANE * _BLK)],
          ringbuf.at[pl.ds(slot * _BLK, _BLK)], sem_pair[slot]).wait()

    def extract(sub_v, src_buf, slot, bdyn, ext):
      sub16 = sub_v[pl.ds(bdyn * _BLK, 16)] & (_LANE - 1)
      blk_ids = slot * _BLK + (iota16 & (_BLK - 1))
      dst_base = iota16 * _EMBED + bdyn * (_BLK * _EMBED)

      @pl.loop(0, _EMBED)
      def _(kk):
        lane_k = jnp.full((16,), 0, jnp.int32) + kk
        vals = plsc.load_gather(src_buf, [blk_ids, lane_k, sub16], mask=m8)
        plsc.store_scatter(ext, [dst_base + kk], vals, mask=m8)

    sems_e = (sem0, sem1)
    sems_w = (sem2, sem3)
    for r in range(_RING):
      fire(emb_hbm, basee_v, ring_e, sems_e, jnp.int32(r), r)
      fire(w_hbm, basew_v, ring_w, sems_w, jnp.int32(r), r)

    @pl.loop(0, _NBLK // _RING)
    def _(phase):
      for r in range(_RING):
        b = phase * _RING + r
        drain(ring_e, sems_e, r)
        extract(sube_v, ring_e, r, b, ext_e)

        @pl.when(phase < _NBLK // _RING - 1)
        def _():
          fire(emb_hbm, basee_v, ring_e, sems_e, b + _RING, r)
        drain(ring_w, sems_w, r)
        extract(subw_v, ring_w, r, b, ext_w)

        @pl.when(phase < _NBLK // _RING - 1)
        def _():
          fire(w_hbm, basew_v, ring_w, sems_w, b + _RING, r)

    w_e = pltpu.make_async_copy(
        ext_e, out_e.at[pl.ds(base_r * _EMBED, _R_TILE * _EMBED)], semo)
    w_e.start()
    w_w = pltpu.make_async_copy(
        ext_w, out_w.at[pl.ds(base_r * _EMBED, _R_TILE * _EMBED)], semo)
    w_w.start()

    samp_bases = bases_v[...]
    for j in range(_BLK):
      bj = pl.multiple_of(
          jnp.sum(jnp.where(iota16 == j, samp_bases, 0), axis=0), _LANE)
      pltpu.make_async_copy(
          w_hbm.at[:, pl.ds(bj, _LANE)], ring_w.at[j], semsmp).start()
    pltpu.make_async_copy(
        emb_hbm.at[:, pl.ds(0, _LANE * _BLK)],
        ring_w.at[pl.ds(0, _BLK)], semsmp).wait()
    sub16 = subs_v[...] & (_LANE - 1)
    blk_ids = iota16 & (_BLK - 1)
    dst_base = iota16 * _EMBED

    @pl.loop(0, _EMBED)
    def _(kk):
      lane_k = jnp.full((16,), 0, jnp.int32) + kk
      vals = plsc.load_gather(ring_w, [blk_ids, lane_k, sub16], mask=m8)
      plsc.store_scatter(ext_s, [dst_base + kk], vals, mask=m8)
    w_s = pltpu.make_async_copy(
        ext_s,
        out_w.at[pl.ds((_BATCH + wid * _BLK) * _EMBED, _BLK * _EMBED)], semo)
    w_s.start()

    for cpy in bias_copies:
      cpy.wait()
    w_b = pltpu.make_async_copy(
        rows_b, out_b.at[pl.ds(base_bb, _BB_TILE)], semo)
    w_b.start()
    w_e.wait()
    w_w.wait()
    w_s.wait()
    w_b.wait()

  return k(emb, w, biases, base_e, sub_e, base_w, sub_w, base_s, sub_s, idx_b)


def _tc_loss_body(embp_ref, wp_ref, bw_ref, bs_ref, labw_ref, samp_ref,
                  swt_ref, out_ref):
  embp = embp_ref[...]
  wp = wp_ref[...]
  bw = bw_ref[...]
  labs = labw_ref[...].astype(jnp.float32)
  samp = samp_ref[...].astype(jnp.float32)
  samp_b = bs_ref[...]
  swt = swt_ref[...]

  logv = jnp.log(float(_VOCAB) + 1.0)
  q_true = (jnp.log(labs + 2.0) - jnp.log(labs + 1.0)) / logv
  q_samp = (jnp.log(samp + 2.0) - jnp.log(samp + 1.0)) / logv

  i0 = lax.broadcasted_iota(jnp.int32, (128, 8), 0)
  i1 = lax.broadcasted_iota(jnp.int32, (128, 8), 1)
  m = jnp.where(i0 // _EMBED == i1, 1.0, 0.0)
  tl8 = lax.dot_general(embp * wp, m, (((1,), (0,)), ((), ())),
                        preferred_element_type=jnp.float32)
  tlw = tl8 + bw - jnp.log(_NUM_SAMPLED * q_true)

  zeros = jnp.zeros((_EMBED, _NUM_SAMPLED), jnp.float32)
  wbig = jnp.concatenate(
      [jnp.concatenate([swt if jj == j else zeros for jj in range(_PACK)],
                       axis=1) for j in range(_PACK)], axis=0)
  sl = lax.dot_general(embp, wbig, (((1,), (0,)), ((), ())),
                       preferred_element_type=jnp.float32)
  corr = samp_b - jnp.log(_NUM_SAMPLED * q_samp)
  sl = sl + jnp.concatenate([corr] * _PACK, axis=1)

  total = jnp.sum(jax.nn.softplus(-tlw)) + jnp.sum(jax.nn.softplus(sl))
  out_ref[...] = jnp.reshape(total / float(_BATCH), (1, 1))


def kernel(inputs, train_labels, embeddings, nce_weights, nce_biases):
  idx_e = inputs.astype(jnp.int32)
  labels = train_labels[:, 0].astype(jnp.int32)
  skey = jax.random.key(12345)
  sampled = jax.random.randint(skey, (_NUM_SAMPLED,), 0, _VOCAB).astype(
      jnp.int32)
  samp_padded = jnp.zeros((_SAMP_PAD,), jnp.int32).at[0:_NUM_SAMPLED].set(
      sampled)
  idx_b = jnp.concatenate([labels, samp_padded])

  base_e = (idx_e // _LANE) * _LANE
  sub_e = idx_e % _LANE
  base_w = (labels // _LANE) * _LANE
  sub_w = labels % _LANE
  base_s = (samp_padded // _LANE) * _LANE
  sub_s = samp_padded % _LANE

  embed_flat, w_flat, b_vals = _sc_gather(
      embeddings.T, nce_weights.T, nce_biases,
      base_e, sub_e, base_w, sub_w, base_s, sub_s, idx_b)
  embed = embed_flat.reshape(_BATCH, _EMBED)

  embp = embed_flat.reshape(_BATCH * _EMBED // 128, 128)
  wp_true = w_flat[: _BATCH * _EMBED].reshape(_BATCH * _EMBED // 128, 128)
  bw = b_vals[:_BATCH].reshape(_BATCH // _PACK, _PACK)
  bs = b_vals[_BATCH:_BATCH + _NUM_SAMPLED].reshape(1, _NUM_SAMPLED)
  labw = labels.reshape(_BATCH // _PACK, _PACK)
  swt = w_flat[_BATCH * _EMBED:(_BATCH + _NUM_SAMPLED) * _EMBED].reshape(
      _NUM_SAMPLED, _EMBED).T

  nce_cost = pl.pallas_call(
      _tc_loss_body,
      out_shape=jax.ShapeDtypeStruct((1, 1), jnp.float32),
  )(embp, wp_true, bw, bs, labw, sampled.reshape(1, _NUM_SAMPLED), swt)[0, 0]

  return embed, nce_cost
